# baseline (speedup 1.0000x reference)
import functools

import jax
import jax.numpy as jnp
from jax import lax
from jax.experimental import pallas as pl
from jax.experimental.pallas import tpu as pltpu
from jax.experimental.pallas import tpu_sc as plsc

N = 10000
D = 128
E = 320000
NC = 2
NS = 16
NW = NC * NS
CH = 128
EP = 327680
EPW = EP // NW
NCHUNK = EPW // CH
NP = 10240
RPT = NP // NS
ZCH = 128


def _sc_aggregate(h, src, dst):
    mesh = plsc.VectorSubcoreMesh(core_axis_name="c", subcore_axis_name="s")

    @functools.partial(
        pl.kernel,
        mesh=mesh,
        out_type=[
            jax.ShapeDtypeStruct((NP, D), jnp.float32),
            jax.ShapeDtypeStruct((NP, D), jnp.float32),
        ],
        scratch_types=[
            pltpu.VMEM((EPW,), jnp.int32),
            pltpu.VMEM((CH,), jnp.int32),
            pltpu.VMEM((CH,), jnp.int32),
            pltpu.VMEM((CH, D), jnp.float32),
            pltpu.VMEM((CH, D), jnp.float32),
            pltpu.VMEM_SHARED((NP, D), jnp.float32),
            pltpu.SemaphoreType.DMA,
            pltpu.SemaphoreType.DMA,
            pltpu.SemaphoreType.DMA,
            pltpu.SemaphoreType.DMA,
        ],
    )
    def agg_kernel(h_hbm, src_hbm, dst_hbm, out0, out1,
                   sidx, didx0, didx1, rows0, rows1, acc,
                   sem0, sem1, dsem0, dsem1):
        cid = lax.axis_index("c")
        sid = lax.axis_index("s")
        wid = cid * NS + sid

        pltpu.sync_copy(src_hbm.at[pl.ds(wid * EPW, EPW)], sidx)

        z = jnp.zeros((16,), jnp.float32)

        def zstore(i, _):
            r = i // (D // 16)
            k = i % (D // 16)
            rows0[r, pl.ds(k * 16, 16)] = z
            return 0

        lax.fori_loop(0, ZCH * (D // 16), zstore, 0)

        def zcopy(j, _):
            pltpu.sync_copy(rows0, acc.at[pl.ds(sid * RPT + j * ZCH, ZCH)])
            return 0

        lax.fori_loop(0, RPT // ZCH, zcopy, 0)
        plsc.subcore_barrier()

        cbase = wid * NCHUNK

        def g(j, buf, sem):
            return pltpu.async_copy(h_hbm.at[sidx.at[pl.ds(j * CH, CH)]], buf, sem)

        def gw(j, buf, sem):
            pltpu.make_async_copy(h_hbm.at[sidx.at[pl.ds(j * CH, CH)]], buf, sem).wait()

        def dl(j, buf, sem):
            return pltpu.async_copy(dst_hbm.at[cbase + j], buf, sem)

        def dw(j, buf, sem):
            pltpu.make_async_copy(dst_hbm.at[cbase + j], buf, sem).wait()

        g(0, rows0, sem0)
        dl(0, didx0, dsem0)
        g(1, rows1, sem1)
        dl(1, didx1, dsem1)

        def body(i, _):
            j0 = 2 * i
            gw(j0, rows0, sem0)
            dw(j0, didx0, dsem0)
            pltpu.sync_copy(rows0, acc.at[didx0], add=True)

            @pl.when(j0 + 2 < NCHUNK)
            def _():
                g(j0 + 2, rows0, sem0)
                dl(j0 + 2, didx0, dsem0)

            j1 = j0 + 1
            gw(j1, rows1, sem1)
            dw(j1, didx1, dsem1)
            pltpu.sync_copy(rows1, acc.at[didx1], add=True)

            @pl.when(j1 + 2 < NCHUNK)
            def _():
                g(j1 + 2, rows1, sem1)
                dl(j1 + 2, didx1, dsem1)

            return 0

        lax.fori_loop(0, NCHUNK // 2, body, 0)
        plsc.subcore_barrier()

        @pl.when(cid == 0)
        def _():
            pltpu.sync_copy(acc.at[pl.ds(sid * RPT, RPT)],
                            out0.at[pl.ds(sid * RPT, RPT)])

        @pl.when(cid == 1)
        def _():
            pltpu.sync_copy(acc.at[pl.ds(sid * RPT, RPT)],
                            out1.at[pl.ds(sid * RPT, RPT)])

    return agg_kernel(h, src, dst)


_BR = 1000


def _tc_linear(x, W, b):
    def body(x_ref, w_ref, b_ref, o_ref):
        o_ref[...] = (
            jnp.dot(x_ref[...], w_ref[...], preferred_element_type=jnp.float32)
            + b_ref[...]
        )

    return pl.pallas_call(
        body,
        grid=(N // _BR,),
        in_specs=[
            pl.BlockSpec((_BR, D), lambda i: (i, 0)),
            pl.BlockSpec((D, D), lambda i: (0, 0)),
            pl.BlockSpec((1, D), lambda i: (0, 0)),
        ],
        out_specs=pl.BlockSpec((_BR, D), lambda i: (i, 0)),
        out_shape=jax.ShapeDtypeStruct((N, D), jnp.float32),
    )(x, W, b.reshape(1, D))


def _tc_add_relu_linear(p0, p1, W, b):
    def body(p0_ref, p1_ref, w_ref, b_ref, o_ref):
        hloc = jnp.maximum(p0_ref[...] + p1_ref[...], 0.0)
        o_ref[...] = (
            jnp.dot(hloc, w_ref[...], preferred_element_type=jnp.float32)
            + b_ref[...]
        )

    return pl.pallas_call(
        body,
        grid=(N // _BR,),
        in_specs=[
            pl.BlockSpec((_BR, D), lambda i: (i, 0)),
            pl.BlockSpec((_BR, D), lambda i: (i, 0)),
            pl.BlockSpec((D, D), lambda i: (0, 0)),
            pl.BlockSpec((1, D), lambda i: (0, 0)),
        ],
        out_specs=pl.BlockSpec((_BR, D), lambda i: (i, 0)),
        out_shape=jax.ShapeDtypeStruct((N, D), jnp.float32),
    )(p0, p1, W, b.reshape(1, D))


def _tc_add_relu_norm(p0, p1):
    def body(p0_ref, p1_ref, o_ref):
        y = jnp.maximum(p0_ref[...] + p1_ref[...], 0.0)
        nrm = jnp.sqrt(jnp.sum(y * y, axis=-1, keepdims=True))
        o_ref[...] = y / jnp.maximum(nrm, 1e-12)

    return pl.pallas_call(
        body,
        grid=(N // _BR,),
        in_specs=[
            pl.BlockSpec((_BR, D), lambda i: (i, 0)),
            pl.BlockSpec((_BR, D), lambda i: (i, 0)),
        ],
        out_specs=pl.BlockSpec((_BR, D), lambda i: (i, 0)),
        out_shape=jax.ShapeDtypeStruct((N, D), jnp.float32),
    )(p0, p1)


def kernel(x, edge_index, W0, b0, W1, b1):
    pad = EP - E
    src = jnp.concatenate([edge_index[0], jnp.zeros((pad,), jnp.int32)])
    dst = jnp.concatenate(
        [edge_index[1], jnp.full((pad,), NP - 1, jnp.int32)]).reshape(EP // CH, CH)
    h1 = _tc_linear(x, W0, b0)
    a0, a1 = _sc_aggregate(h1, src, dst)
    h2 = _tc_add_relu_linear(a0, a1, W1, b1)
    c0, c1 = _sc_aggregate(h2, src, dst)
    return _tc_add_relu_norm(c0, c1)


# speedup vs baseline: 1.7804x; 1.7804x over previous
import functools

import jax
import jax.numpy as jnp
from jax import lax
from jax.experimental import pallas as pl
from jax.experimental.pallas import tpu as pltpu
from jax.experimental.pallas import tpu_sc as plsc

N = 10000
D = 128
E = 320000
NC = 2
NS = 16
NW = NC * NS
CH = 80
NCHUNK = 126
EPW = NCHUNK * CH
EP = EPW * NW
NP = 10240
RPT = NP // NS
ZCH = CH


def _sc_aggregate(h, src, dst):
    mesh = plsc.VectorSubcoreMesh(core_axis_name="c", subcore_axis_name="s")

    @functools.partial(
        pl.kernel,
        mesh=mesh,
        out_type=[
            jax.ShapeDtypeStruct((NP, D), jnp.float32),
            jax.ShapeDtypeStruct((NP, D), jnp.float32),
        ],
        scratch_types=[
            pltpu.VMEM((CH,), jnp.int32),
            pltpu.VMEM((CH,), jnp.int32),
            pltpu.VMEM((CH,), jnp.int32),
            pltpu.VMEM((CH,), jnp.int32),
            pltpu.VMEM((CH, D), jnp.float32),
            pltpu.VMEM((CH, D), jnp.float32),
            pltpu.VMEM_SHARED((NP, D), jnp.float32),
            pltpu.SemaphoreType.DMA,
            pltpu.SemaphoreType.DMA,
            pltpu.SemaphoreType.DMA,
            pltpu.SemaphoreType.DMA,
            pltpu.SemaphoreType.DMA,
            pltpu.SemaphoreType.DMA,
        ],
    )
    def agg_kernel(h_hbm, src_hbm, dst_hbm, out0, out1,
                   sidx0, sidx1, didx0, didx1, rows0, rows1, acc,
                   sem0, sem1, ssem0, ssem1, dsem0, dsem1):
        cid = lax.axis_index("c")
        sid = lax.axis_index("s")
        wid = cid * NS + sid

        z = jnp.zeros((16,), jnp.float32)

        def zstore(i, _):
            r = i // (D // 16)
            k = i % (D // 16)
            rows0[r, pl.ds(k * 16, 16)] = z
            return 0

        lax.fori_loop(0, ZCH * (D // 16), zstore, 0)

        def zcopy(j, _):
            pltpu.sync_copy(rows0, acc.at[pl.ds(sid * RPT + j * ZCH, ZCH)])
            return 0

        lax.fori_loop(0, RPT // ZCH, zcopy, 0)
        plsc.subcore_barrier()

        base0 = wid * EPW
        sd = (sidx0, didx0, ssem0, dsem0, rows0, sem0)
        sd1 = (sidx1, didx1, ssem1, dsem1, rows1, sem1)

        def iload(j, bufs):
            si, di, ss, ds_, _, _ = bufs
            pltpu.async_copy(src_hbm.at[pl.ds(base0 + j * CH, CH)], si, ss)
            pltpu.async_copy(dst_hbm.at[pl.ds(base0 + j * CH, CH)], di, ds_)

        def iwait(j, bufs):
            si, di, ss, ds_, _, _ = bufs
            pltpu.make_async_copy(src_hbm.at[pl.ds(base0 + j * CH, CH)], si, ss).wait()
            pltpu.make_async_copy(dst_hbm.at[pl.ds(base0 + j * CH, CH)], di, ds_).wait()

        def gstart(bufs):
            si, _, _, _, rw, gs = bufs
            pltpu.async_copy(h_hbm.at[si], rw, gs)

        def gwait(bufs):
            si, _, _, _, rw, gs = bufs
            pltpu.make_async_copy(h_hbm.at[si], rw, gs).wait()

        def scatter(bufs):
            _, di, _, _, rw, _ = bufs
            pltpu.sync_copy(rw, acc.at[di], add=True)

        iload(0, sd)
        iwait(0, sd)
        gstart(sd)
        iload(1, sd1)

        def body(i, _):
            j0 = 2 * i
            iwait(j0 + 1, sd1)
            gstart(sd1)
            gwait(sd)
            scatter(sd)

            @pl.when(j0 + 2 < NCHUNK)
            def _():
                iload(j0 + 2, sd)

            j1 = j0 + 1

            @pl.when(j1 + 1 < NCHUNK)
            def _():
                iwait(j1 + 1, sd)
                gstart(sd)
            gwait(sd1)
            scatter(sd1)

            @pl.when(j1 + 2 < NCHUNK)
            def _():
                iload(j1 + 2, sd1)

            return 0

        lax.fori_loop(0, NCHUNK // 2, body, 0)
        plsc.subcore_barrier()

        @pl.when(cid == 0)
        def _():
            pltpu.sync_copy(acc.at[pl.ds(sid * RPT, RPT)],
                            out0.at[pl.ds(sid * RPT, RPT)])

        @pl.when(cid == 1)
        def _():
            pltpu.sync_copy(acc.at[pl.ds(sid * RPT, RPT)],
                            out1.at[pl.ds(sid * RPT, RPT)])

    return agg_kernel(h, src, dst)


_BR = 1000


def _tc_linear(x, W, b):
    def body(x_ref, w_ref, b_ref, o_ref):
        o_ref[...] = (
            jnp.dot(x_ref[...], w_ref[...], preferred_element_type=jnp.float32)
            + b_ref[...]
        )

    return pl.pallas_call(
        body,
        grid=(N // _BR,),
        in_specs=[
            pl.BlockSpec((_BR, D), lambda i: (i, 0)),
            pl.BlockSpec((D, D), lambda i: (0, 0)),
            pl.BlockSpec((1, D), lambda i: (0, 0)),
        ],
        out_specs=pl.BlockSpec((_BR, D), lambda i: (i, 0)),
        out_shape=jax.ShapeDtypeStruct((N, D), jnp.float32),
    )(x, W, b.reshape(1, D))


def _tc_add_relu_linear(p0, p1, W, b):
    def body(p0_ref, p1_ref, w_ref, b_ref, o_ref):
        hloc = jnp.maximum(p0_ref[...] + p1_ref[...], 0.0)
        o_ref[...] = (
            jnp.dot(hloc, w_ref[...], preferred_element_type=jnp.float32)
            + b_ref[...]
        )

    return pl.pallas_call(
        body,
        grid=(N // _BR,),
        in_specs=[
            pl.BlockSpec((_BR, D), lambda i: (i, 0)),
            pl.BlockSpec((_BR, D), lambda i: (i, 0)),
            pl.BlockSpec((D, D), lambda i: (0, 0)),
            pl.BlockSpec((1, D), lambda i: (0, 0)),
        ],
        out_specs=pl.BlockSpec((_BR, D), lambda i: (i, 0)),
        out_shape=jax.ShapeDtypeStruct((N, D), jnp.float32),
    )(p0, p1, W, b.reshape(1, D))


def _tc_add_relu_norm(p0, p1):
    def body(p0_ref, p1_ref, o_ref):
        y = jnp.maximum(p0_ref[...] + p1_ref[...], 0.0)
        nrm = jnp.sqrt(jnp.sum(y * y, axis=-1, keepdims=True))
        o_ref[...] = y / jnp.maximum(nrm, 1e-12)

    return pl.pallas_call(
        body,
        grid=(N // _BR,),
        in_specs=[
            pl.BlockSpec((_BR, D), lambda i: (i, 0)),
            pl.BlockSpec((_BR, D), lambda i: (i, 0)),
        ],
        out_specs=pl.BlockSpec((_BR, D), lambda i: (i, 0)),
        out_shape=jax.ShapeDtypeStruct((N, D), jnp.float32),
    )(p0, p1)


def kernel(x, edge_index, W0, b0, W1, b1):
    pad = EP - E
    src = jnp.concatenate([edge_index[0], jnp.zeros((pad,), jnp.int32)])
    dst = jnp.concatenate([edge_index[1], jnp.full((pad,), NP - 1, jnp.int32)])
    h1 = _tc_linear(x, W0, b0)
    a0, a1 = _sc_aggregate(h1, src, dst)
    h2 = _tc_add_relu_linear(a0, a1, W1, b1)
    c0, c1 = _sc_aggregate(h2, src, dst)
    return _tc_add_relu_norm(c0, c1)
